# baseline (speedup 1.0000x reference)
import functools

import jax
import jax.numpy as jnp
from jax import lax
from jax.experimental import pallas as pl
from jax.experimental.pallas import tpu as pltpu
from jax.experimental.pallas import tpu_sc as plsc

N_EXP = 16
L = 16
NW = 32
BLOCK_M = 2048
N_CHUNKS = 4


def _logits_body(w_ref, x_ref, b_ref, out_ref):
    out_ref[...] = (
        lax.dot_general(
            w_ref[...], x_ref[...],
            dimension_numbers=(((1,), (1,)), ((), ())),
            preferred_element_type=jnp.float32,
        )
        + b_ref[...]
    )


def _route_body(rows_per_tile, logits_hbm, w_hbm, i_hbm, logits_v, w_v, i_v):
    wid = lax.axis_index("s") * 2 + lax.axis_index("c")
    base = wid * rows_per_tile
    pltpu.sync_copy(logits_hbm.at[:, pl.ds(base, rows_per_tile)], logits_v)

    def group(g, carry):
        col = g * L
        vs = [logits_v[e, pl.ds(col, L)] for e in range(N_EXP)]
        m1 = vs[0]
        i1 = jnp.zeros((L,), jnp.int32)
        m2 = jnp.full((L,), -jnp.inf, jnp.float32)
        i2 = jnp.zeros((L,), jnp.int32)
        for e in range(1, N_EXP):
            v = vs[e]
            ei = jnp.full((L,), e, jnp.int32)
            gt1 = v > m1
            gt2 = v > m2
            m2 = jnp.where(gt1, m1, jnp.where(gt2, v, m2))
            i2 = jnp.where(gt1, i1, jnp.where(gt2, ei, i2))
            m1 = jnp.where(gt1, v, m1)
            i1 = jnp.where(gt1, ei, i1)
        e2 = jnp.exp(m2 - m1)
        denom = e2 + jnp.float32(1.0 + 1e-8)
        w1 = jnp.float32(1.0) / denom
        w2 = e2 / denom
        w_v[0, pl.ds(col, L)] = w1
        w_v[1, pl.ds(col, L)] = w2
        i_v[0, pl.ds(col, L)] = i1
        i_v[1, pl.ds(col, L)] = i2
        return carry

    lax.fori_loop(0, rows_per_tile // L, group, 0)
    pltpu.sync_copy(w_v, w_hbm.at[:, pl.ds(base, rows_per_tile)])
    pltpu.sync_copy(i_v, i_hbm.at[:, pl.ds(base, rows_per_tile)])


def kernel(x, gate_weight, expert_bias):
    n_tokens, d_model = x.shape
    bias = expert_bias.reshape(N_EXP, 1)
    chunk = n_tokens // N_CHUNKS
    rows_per_tile = chunk // NW

    def make_matmul(c):
        base_blk = c * (chunk // BLOCK_M)
        return pl.pallas_call(
            _logits_body,
            grid=(chunk // BLOCK_M,),
            in_specs=[
                pl.BlockSpec((N_EXP, d_model), lambda i: (0, 0)),
                pl.BlockSpec((BLOCK_M, d_model), lambda i: (base_blk + i, 0)),
                pl.BlockSpec((N_EXP, 1), lambda i: (0, 0)),
            ],
            out_specs=pl.BlockSpec((N_EXP, BLOCK_M), lambda i: (0, i)),
            out_shape=jax.ShapeDtypeStruct((N_EXP, chunk), jnp.float32),
            compiler_params=pltpu.CompilerParams(
                dimension_semantics=("arbitrary",),
            ),
        )

    route = pl.kernel(
        functools.partial(_route_body, rows_per_tile),
        out_type=[
            jax.ShapeDtypeStruct((2, chunk), jnp.float32),
            jax.ShapeDtypeStruct((2, chunk), jnp.int32),
        ],
        mesh=plsc.VectorSubcoreMesh(core_axis_name="c", subcore_axis_name="s"),
        scratch_types=[
            pltpu.VMEM((N_EXP, rows_per_tile), jnp.float32),
            pltpu.VMEM((2, rows_per_tile), jnp.float32),
            pltpu.VMEM((2, rows_per_tile), jnp.int32),
        ],
    )

    w_parts, i_parts = [], []
    for c in range(N_CHUNKS):
        logits_t = make_matmul(c)(gate_weight, x, bias)
        w_t, i_t = route(logits_t)
        w_parts.append(w_t)
        i_parts.append(i_t)
    w = jnp.concatenate(w_parts, axis=1).T
    i = jnp.concatenate(i_parts, axis=1).T
    return (w, i)


# speedup vs baseline: 1.7015x; 1.7015x over previous
import jax
import jax.numpy as jnp
from jax import lax
from jax.experimental import pallas as pl
from jax.experimental.pallas import tpu as pltpu

N_EXP = 16
BLOCK_M = 2048


def _router_body(w_ref, x_ref, b_ref, w_out_ref, i_out_ref):
    logits = (
        lax.dot_general(
            w_ref[...], x_ref[...],
            dimension_numbers=(((1,), (1,)), ((), ())),
            preferred_element_type=jnp.float32,
        )
        + b_ref[...]
    )
    eidx = jax.lax.broadcasted_iota(jnp.int32, logits.shape, 0)
    neg_inf = jnp.float32(-jnp.inf)

    m1 = jnp.max(logits, axis=0, keepdims=True)
    i1 = jnp.min(jnp.where(logits == m1, eidx, N_EXP), axis=0, keepdims=True)
    masked = jnp.where(eidx == i1, neg_inf, logits)
    m2 = jnp.max(masked, axis=0, keepdims=True)
    i2 = jnp.min(jnp.where(masked == m2, eidx, N_EXP), axis=0, keepdims=True)

    e2 = jnp.exp(m2 - m1)
    denom = e2 + jnp.float32(1.0 + 1e-8)
    w1 = jnp.float32(1.0) / denom
    w2 = e2 / denom
    w_out_ref[...] = jnp.concatenate([w1, w2], axis=0)
    i_out_ref[...] = jnp.concatenate([i1, i2], axis=0)


def kernel(x, gate_weight, expert_bias):
    n_tokens, d_model = x.shape
    bias = expert_bias.reshape(N_EXP, 1)

    w_t, i_t = pl.pallas_call(
        _router_body,
        grid=(n_tokens // BLOCK_M,),
        in_specs=[
            pl.BlockSpec((N_EXP, d_model), lambda i: (0, 0)),
            pl.BlockSpec((BLOCK_M, d_model), lambda i: (i, 0)),
            pl.BlockSpec((N_EXP, 1), lambda i: (0, 0)),
        ],
        out_specs=[
            pl.BlockSpec((2, BLOCK_M), lambda i: (0, i)),
            pl.BlockSpec((2, BLOCK_M), lambda i: (0, i)),
        ],
        out_shape=[
            jax.ShapeDtypeStruct((2, n_tokens), jnp.float32),
            jax.ShapeDtypeStruct((2, n_tokens), jnp.int32),
        ],
        compiler_params=pltpu.CompilerParams(
            dimension_semantics=("arbitrary",),
        ),
    )(gate_weight, x, bias)
    return (w_t.T, i_t.T)


# speedup vs baseline: 1.7880x; 1.0508x over previous
import jax
import jax.numpy as jnp
from jax import lax
from jax.experimental import pallas as pl
from jax.experimental.pallas import tpu as pltpu

N_EXP = 16
BLOCK_M = 1024


def _router_body(w_ref, x_ref, b_ref, w_out_ref, i_out_ref):
    logits = (
        lax.dot_general(
            w_ref[...], x_ref[...],
            dimension_numbers=(((1,), (1,)), ((), ())),
            preferred_element_type=jnp.float32,
        )
        + b_ref[...]
    )
    eidx = jax.lax.broadcasted_iota(jnp.int32, logits.shape, 0)
    neg_inf = jnp.float32(-jnp.inf)

    m1 = jnp.max(logits, axis=0, keepdims=True)
    i1 = jnp.min(jnp.where(logits == m1, eidx, N_EXP), axis=0, keepdims=True)
    masked = jnp.where(eidx == i1, neg_inf, logits)
    m2 = jnp.max(masked, axis=0, keepdims=True)
    i2 = jnp.min(jnp.where(masked == m2, eidx, N_EXP), axis=0, keepdims=True)

    e2 = jnp.exp(m2 - m1)
    denom = e2 + jnp.float32(1.0 + 1e-8)
    w1 = jnp.float32(1.0) / denom
    w2 = e2 / denom
    w_out_ref[...] = jnp.concatenate([w1, w2], axis=0)
    i_out_ref[...] = jnp.concatenate([i1, i2], axis=0)


def kernel(x, gate_weight, expert_bias):
    n_tokens, d_model = x.shape
    bias = expert_bias.reshape(N_EXP, 1)

    w_t, i_t = pl.pallas_call(
        _router_body,
        grid=(n_tokens // BLOCK_M,),
        in_specs=[
            pl.BlockSpec((N_EXP, d_model), lambda i: (0, 0)),
            pl.BlockSpec((BLOCK_M, d_model), lambda i: (i, 0)),
            pl.BlockSpec((N_EXP, 1), lambda i: (0, 0)),
        ],
        out_specs=[
            pl.BlockSpec((2, BLOCK_M), lambda i: (0, i)),
            pl.BlockSpec((2, BLOCK_M), lambda i: (0, i)),
        ],
        out_shape=[
            jax.ShapeDtypeStruct((2, n_tokens), jnp.float32),
            jax.ShapeDtypeStruct((2, n_tokens), jnp.int32),
        ],
        compiler_params=pltpu.CompilerParams(
            dimension_semantics=("arbitrary",),
        ),
    )(gate_weight, x, bias)
    return (w_t.T, i_t.T)
